# baseline (speedup 1.0000x reference)
import functools

import jax
import jax.numpy as jnp
from jax import lax
from jax.experimental import pallas as pl
from jax.experimental.pallas import tpu as pltpu
from jax.experimental.pallas import tpu_sc as plsc

B, S, D = 4, 8192, 1024
TOTAL = B * S * D
POS_TOTAL = S * D
NW = 32
ROWS_W = S // NW
CHR = 16
CHF = CHR * D
NK = ROWS_W // CHR
NT = NK * B
LANES = 16

_mesh = plsc.VectorSubcoreMesh(core_axis_name="c", subcore_axis_name="s")


@functools.partial(
    pl.kernel,
    mesh=_mesh,
    out_type=jax.ShapeDtypeStruct((TOTAL,), jnp.float32),
    scratch_types=[
        pltpu.VMEM((CHF,), jnp.float32),
        pltpu.VMEM((CHF,), jnp.float32),
        pltpu.VMEM((CHF,), jnp.float32),
        pltpu.VMEM((CHF,), jnp.float32),
        pltpu.VMEM((CHF,), jnp.float32),
        pltpu.VMEM((CHF,), jnp.float32),
        pltpu.SemaphoreType.DMA,
        pltpu.SemaphoreType.DMA,
        pltpu.SemaphoreType.DMA,
        pltpu.SemaphoreType.DMA,
        pltpu.SemaphoreType.DMA,
        pltpu.SemaphoreType.DMA,
        pltpu.SemaphoreType.DMA,
        pltpu.SemaphoreType.DMA,
        pltpu.SemaphoreType.DMA,
        pltpu.SemaphoreType.DMA,
    ],
)
def _sc_add(x_hbm, pos_hbm, out_hbm,
            a0, a1, a2, a3, p0, p1,
            sx0, sx1, sx2, sx3, sp0, sp1, so0, so1, so2, so3):
    accs, poss = (a0, a1, a2, a3), (p0, p1)
    sxs, sps, sos = (sx0, sx1, sx2, sx3), (sp0, sp1), (so0, so1, so2, so3)

    wid = lax.axis_index("s") * 2 + lax.axis_index("c")
    row0 = wid * ROWS_W

    def x_slice(t):
        k, b = divmod(t, B)
        off = b * POS_TOTAL + (row0 + k * CHR) * D
        return pl.ds(off, CHF)

    def pos_slice(k):
        return pl.ds((row0 + k * CHR) * D, CHF)

    def x_copy(t):
        return pltpu.make_async_copy(x_hbm.at[x_slice(t)], accs[t % 4], sxs[t % 4])

    def pos_copy(k):
        return pltpu.make_async_copy(pos_hbm.at[pos_slice(k)], poss[k % 2], sps[k % 2])

    def out_copy(t):
        return pltpu.make_async_copy(accs[t % 4], out_hbm.at[x_slice(t)], sos[t % 4])

    pos_copy(0).start()
    x_copy(0).start()
    x_copy(1).start()

    for t in range(NT):
        k, b = divmod(t, B)
        if b == 1 and k + 1 < NK:
            pos_copy(k + 1).start()
        if t >= 2:
            out_copy(t - 2).wait()
        if t + 2 < NT:
            x_copy(t + 2).start()
        if b == 0:
            pos_copy(k).wait()
        x_copy(t).wait()

        acc, pos = accs[t % 4], poss[k % 2]

        @plsc.parallel_loop(0, CHF // LANES, unroll=8)
        def _(i):
            sl = pl.ds(i * LANES, LANES)
            plsc.addupdate(acc.at[sl], pos[sl])

        out_copy(t).start()

    out_copy(NT - 2).wait()
    out_copy(NT - 1).wait()


def kernel(x, pos_table):
    out = _sc_add(x.reshape(-1), pos_table.reshape(-1))
    return out.reshape(B, S, D)


# speedup vs baseline: 2.9765x; 2.9765x over previous
import functools

import jax
import jax.numpy as jnp
from jax import lax
from jax.experimental import pallas as pl
from jax.experimental.pallas import tpu as pltpu
from jax.experimental.pallas import tpu_sc as plsc

B, S, D = 4, 8192, 1024
NW = 32
ROWS_W = S // NW
CHR = 16
NK = ROWS_W // CHR
NT = NK * B
LANES = 16
VPR = D // LANES

_mesh = plsc.VectorSubcoreMesh(core_axis_name="c", subcore_axis_name="s")


@functools.partial(
    pl.kernel,
    mesh=_mesh,
    out_type=jax.ShapeDtypeStruct((B, S, D), jnp.float32),
    scratch_types=[
        pltpu.VMEM((CHR, D), jnp.float32),
        pltpu.VMEM((CHR, D), jnp.float32),
        pltpu.VMEM((CHR, D), jnp.float32),
        pltpu.VMEM((CHR, D), jnp.float32),
        pltpu.VMEM((CHR, D), jnp.float32),
        pltpu.VMEM((CHR, D), jnp.float32),
        pltpu.SemaphoreType.DMA,
        pltpu.SemaphoreType.DMA,
        pltpu.SemaphoreType.DMA,
        pltpu.SemaphoreType.DMA,
        pltpu.SemaphoreType.DMA,
        pltpu.SemaphoreType.DMA,
        pltpu.SemaphoreType.DMA,
        pltpu.SemaphoreType.DMA,
        pltpu.SemaphoreType.DMA,
        pltpu.SemaphoreType.DMA,
    ],
    compiler_params=pltpu.CompilerParams(use_tc_tiling_on_sc=True),
)
def _sc_add(x_hbm, pos_hbm, out_hbm,
            a0, a1, a2, a3, p0, p1,
            sx0, sx1, sx2, sx3, sp0, sp1, so0, so1, so2, so3):
    accs, poss = (a0, a1, a2, a3), (p0, p1)
    sxs, sps, sos = (sx0, sx1, sx2, sx3), (sp0, sp1), (so0, so1, so2, so3)

    wid = lax.axis_index("s") * 2 + lax.axis_index("c")
    row0 = wid * ROWS_W

    def rows(k):
        return pl.ds(row0 + k * CHR, CHR)

    def x_copy(t):
        k, b = divmod(t, B)
        return pltpu.make_async_copy(x_hbm.at[b, rows(k), :], accs[t % 4], sxs[t % 4])

    def pos_copy(k):
        return pltpu.make_async_copy(pos_hbm.at[rows(k), :], poss[k % 2], sps[k % 2])

    def out_copy(t):
        k, b = divmod(t, B)
        return pltpu.make_async_copy(accs[t % 4], out_hbm.at[b, rows(k), :], sos[t % 4])

    pos_copy(0).start()
    x_copy(0).start()
    x_copy(1).start()

    for t in range(NT):
        k, b = divmod(t, B)
        if b == 1 and k + 1 < NK:
            pos_copy(k + 1).start()
        if t >= 2:
            out_copy(t - 2).wait()
        if t + 2 < NT:
            x_copy(t + 2).start()
        if b == 0:
            pos_copy(k).wait()
        x_copy(t).wait()

        acc, pos = accs[t % 4], poss[k % 2]

        @plsc.parallel_loop(0, CHR * VPR, unroll=8)
        def _(i):
            r = i // VPR
            sl = pl.ds((i % VPR) * LANES, LANES)
            plsc.addupdate(acc.at[r, sl], pos[r, sl])

        out_copy(t).start()

    out_copy(NT - 2).wait()
    out_copy(NT - 1).wait()


def kernel(x, pos_table):
    return _sc_add(x, pos_table)


# speedup vs baseline: 3.0068x; 1.0102x over previous
import functools

import jax
import jax.numpy as jnp
from jax import lax
from jax.experimental import pallas as pl
from jax.experimental.pallas import tpu as pltpu
from jax.experimental.pallas import tpu_sc as plsc

B, S, D = 4, 8192, 1024
NW = 32
ROWS_W = S // NW
CHR = 16
NK = ROWS_W // CHR
NT = NK * B
LANES = 16
VPR = D // LANES

_mesh = plsc.VectorSubcoreMesh(core_axis_name="c", subcore_axis_name="s")


@functools.partial(
    pl.kernel,
    mesh=_mesh,
    out_type=jax.ShapeDtypeStruct((B, S, D), jnp.float32),
    scratch_types=[
        pltpu.VMEM((CHR, D), jnp.float32),
        pltpu.VMEM((CHR, D), jnp.float32),
        pltpu.VMEM((CHR, D), jnp.float32),
        pltpu.VMEM((CHR, D), jnp.float32),
        pltpu.VMEM((CHR, D), jnp.float32),
        pltpu.VMEM((CHR, D), jnp.float32),
        pltpu.VMEM((CHR, D), jnp.float32),
        pltpu.SemaphoreType.DMA,
        pltpu.SemaphoreType.DMA,
        pltpu.SemaphoreType.DMA,
        pltpu.SemaphoreType.DMA,
        pltpu.SemaphoreType.DMA,
        pltpu.SemaphoreType.DMA,
        pltpu.SemaphoreType.DMA,
        pltpu.SemaphoreType.DMA,
        pltpu.SemaphoreType.DMA,
        pltpu.SemaphoreType.DMA,
        pltpu.SemaphoreType.DMA,
        pltpu.SemaphoreType.DMA,
    ],
    compiler_params=pltpu.CompilerParams(use_tc_tiling_on_sc=True),
)
def _sc_add(x_hbm, pos_hbm, out_hbm,
            a0, a1, a2, a3, a4, p0, p1,
            sx0, sx1, sx2, sx3, sx4, sp0, sp1, so0, so1, so2, so3, so4):
    accs, poss = (a0, a1, a2, a3, a4), (p0, p1)
    sxs, sps, sos = (sx0, sx1, sx2, sx3, sx4), (sp0, sp1), (so0, so1, so2, so3, so4)

    wid = lax.axis_index("s") * 2 + lax.axis_index("c")
    row0 = wid * ROWS_W

    def rows(k):
        return pl.ds(row0 + k * CHR, CHR)

    def x_copy(t):
        k, b = divmod(t, B)
        return pltpu.make_async_copy(x_hbm.at[b, rows(k), :], accs[t % 5], sxs[t % 5])

    def pos_copy(k):
        return pltpu.make_async_copy(pos_hbm.at[rows(k), :], poss[k % 2], sps[k % 2])

    def out_copy(t):
        k, b = divmod(t, B)
        return pltpu.make_async_copy(accs[t % 5], out_hbm.at[b, rows(k), :], sos[t % 5])

    pos_copy(0).start()
    x_copy(0).start()
    x_copy(1).start()
    x_copy(2).start()

    for t in range(NT):
        k, b = divmod(t, B)
        if b == 1 and k + 1 < NK:
            pos_copy(k + 1).start()
        if t >= 2:
            out_copy(t - 2).wait()
        if t + 3 < NT:
            x_copy(t + 3).start()
        if b == 0:
            pos_copy(k).wait()
        x_copy(t).wait()

        acc, pos = accs[t % 5], poss[k % 2]

        @plsc.parallel_loop(0, CHR * VPR, unroll=8)
        def _(i):
            r = i // VPR
            sl = pl.ds((i % VPR) * LANES, LANES)
            plsc.addupdate(acc.at[r, sl], pos[r, sl])

        out_copy(t).start()

    out_copy(NT - 2).wait()
    out_copy(NT - 1).wait()


def kernel(x, pos_table):
    return _sc_add(x, pos_table)
